# baseline (speedup 1.0000x reference)
import functools

import jax
import jax.numpy as jnp
from jax import lax
from jax.experimental import pallas as pl
from jax.experimental.pallas import tpu as pltpu
from jax.experimental.pallas import tpu_sc as plsc

_E = 8
_T = 32
_H = 4096
_O = 4096
_H_BS = 512
_NC = 2
_NS = 16
_L = 16


def _router_body(x_hbm, gate_hbm, ew_hbm, out_hbm, xv, gv, wv):
    t = lax.axis_index("s") * _NC + lax.axis_index("c")
    pltpu.sync_copy(x_hbm.at[t], xv)
    pltpu.sync_copy(gate_hbm, gv)
    lg = []
    for e in range(_E):
        def body(i, acc, e=e):
            off = i * _L
            return acc + xv[pl.ds(off, _L)] * gv[e, pl.ds(off, _L)]
        acc = lax.fori_loop(0, _H // _L, body,
                            jnp.zeros((_L,), jnp.float32), unroll=8)
        s = acc[0]
        for j in range(1, _L):
            s = s + acc[j]
        lg.append(s)
    m1 = lg[0]
    for e in range(1, _E):
        m1 = jnp.maximum(m1, lg[e])
    i1 = jnp.int32(_E - 1)
    for e in range(_E - 2, -1, -1):
        i1 = jnp.where(lg[e] == m1, jnp.int32(e), i1)
    m2 = jnp.float32(-jnp.inf)
    for e in range(_E):
        keep = (jnp.int32(e) != i1) & (lg[e] > m2)
        m2 = jnp.where(keep, lg[e], m2)
    i2 = jnp.int32(_E - 1)
    for e in range(_E - 2, -1, -1):
        i2 = jnp.where((lg[e] == m2) & (jnp.int32(e) != i1),
                       jnp.int32(e), i2)
    idx = lax.iota(jnp.int32, _L)
    r = jnp.exp(jnp.full((_L,), m2 - m1, jnp.float32))
    w1 = 1.0 / (1.0 + r)
    w2 = 1.0 - w1
    wv[...] = jnp.where(idx == i1, w1, 0.0) + jnp.where(idx == i2, w2, 0.0)
    pltpu.sync_copy(wv, out_hbm.at[t])

    def probe(i, c):
        pltpu.sync_copy(ew_hbm.at[0, t * 128 + i], xv)
        return c
    lax.fori_loop(0, 128, probe, jnp.int32(0))


def _router(xf, gate_t, expert_w):
    mesh = plsc.VectorSubcoreMesh(core_axis_name="c", subcore_axis_name="s")
    k = functools.partial(
        pl.kernel,
        mesh=mesh,
        out_type=jax.ShapeDtypeStruct((_T, _L), jnp.float32),
        scratch_types=[
            pltpu.VMEM((_H,), jnp.float32),
            pltpu.VMEM((_E, _H), jnp.float32),
            pltpu.VMEM((_L,), jnp.float32),
        ],
    )(_router_body)
    return k(xf, gate_t, expert_w)


def _partials_body(x_ref, w_ref, out_ref):
    h = pl.program_id(1)

    @pl.when(h == 0)
    def _init():
        out_ref[...] = jnp.zeros_like(out_ref)

    xs = x_ref[:, pl.ds(h * _H_BS, _H_BS)]
    out_ref[0] += jnp.dot(xs, w_ref[0], preferred_element_type=jnp.float32)


def _combine_body(p_ref, wts_ref, out_ref):
    acc = p_ref[0] * wts_ref[:, 0:1]
    for e in range(1, _E):
        acc += p_ref[e] * wts_ref[:, e:e + 1]
    out_ref[...] = acc


def kernel(x, gate_w, expert_w):
    B_, S_, H = x.shape
    xf = x.reshape(B_ * S_, H)
    wts = _router(xf, gate_w.T, expert_w)
    partials = pl.pallas_call(
        _partials_body,
        grid=(_E, _H // _H_BS),
        in_specs=[
            pl.BlockSpec((_T, _H), lambda e, h: (0, 0)),
            pl.BlockSpec((1, _H_BS, _O), lambda e, h: (e, h, 0)),
        ],
        out_specs=pl.BlockSpec((1, _T, _O), lambda e, h: (e, 0, 0)),
        out_shape=jax.ShapeDtypeStruct((_E, _T, _O), jnp.float32),
        compiler_params=pltpu.CompilerParams(
            dimension_semantics=("arbitrary", "arbitrary")),
    )(xf, expert_w)
    out = pl.pallas_call(
        _combine_body,
        in_specs=[
            pl.BlockSpec((_E, _T, _O), lambda: (0, 0, 0)),
            pl.BlockSpec((_T, _L), lambda: (0, 0)),
        ],
        out_specs=pl.BlockSpec((_T, _O), lambda: (0, 0)),
        out_shape=jax.ShapeDtypeStruct((_T, _O), jnp.float32),
    )(partials, wts)
    return out.reshape(B_, S_, _O)


# speedup vs baseline: 1.0912x; 1.0912x over previous
import functools

import jax
import jax.numpy as jnp
from jax import lax
from jax.experimental import pallas as pl
from jax.experimental.pallas import tpu as pltpu
from jax.experimental.pallas import tpu_sc as plsc

_E = 8
_T = 32
_H = 4096
_O = 4096
_H_BS = 512
_NC = 2
_NS = 16
_L = 16


def _router_body(x_hbm, gate_hbm, ew_hbm, out_hbm, xv, gv, wv):
    t = lax.axis_index("s") * _NC + lax.axis_index("c")
    pltpu.sync_copy(x_hbm.at[t], xv)
    pltpu.sync_copy(gate_hbm, gv)
    lg = []
    for e in range(_E):
        def body(i, acc, e=e):
            off = i * _L
            return acc + xv[pl.ds(off, _L)] * gv[e, pl.ds(off, _L)]
        acc = lax.fori_loop(0, _H // _L, body,
                            jnp.zeros((_L,), jnp.float32), unroll=8)
        s = acc[0]
        for j in range(1, _L):
            s = s + acc[j]
        lg.append(s)
    m1 = lg[0]
    for e in range(1, _E):
        m1 = jnp.maximum(m1, lg[e])
    i1 = jnp.int32(_E - 1)
    for e in range(_E - 2, -1, -1):
        i1 = jnp.where(lg[e] == m1, jnp.int32(e), i1)
    m2 = jnp.float32(-jnp.inf)
    for e in range(_E):
        keep = (jnp.int32(e) != i1) & (lg[e] > m2)
        m2 = jnp.where(keep, lg[e], m2)
    i2 = jnp.int32(_E - 1)
    for e in range(_E - 2, -1, -1):
        i2 = jnp.where((lg[e] == m2) & (jnp.int32(e) != i1),
                       jnp.int32(e), i2)
    idx = lax.iota(jnp.int32, _L)
    r = jnp.exp(jnp.full((_L,), m2 - m1, jnp.float32))
    w1 = 1.0 / (1.0 + r)
    w2 = 1.0 - w1
    wv[...] = jnp.where(idx == i1, w1, 0.0) + jnp.where(idx == i2, w2, 0.0)
    pltpu.sync_copy(wv, out_hbm.at[t])


def _router(xf, gate_t, expert_w):
    mesh = plsc.VectorSubcoreMesh(core_axis_name="c", subcore_axis_name="s")
    k = functools.partial(
        pl.kernel,
        mesh=mesh,
        out_type=jax.ShapeDtypeStruct((_T, _L), jnp.float32),
        scratch_types=[
            pltpu.VMEM((_H,), jnp.float32),
            pltpu.VMEM((_E, _H), jnp.float32),
            pltpu.VMEM((_L,), jnp.float32),
        ],
    )(_router_body)
    return k(xf, gate_t, expert_w)


def _partials_body(x_ref, w_ref, out_ref):
    h = pl.program_id(1)

    @pl.when(h == 0)
    def _init():
        out_ref[...] = jnp.zeros_like(out_ref)

    xs = x_ref[:, pl.ds(h * _H_BS, _H_BS)]
    out_ref[0] += jnp.dot(xs, w_ref[0], preferred_element_type=jnp.float32)


_C_BS = 512


def _combine_body(p_ref, wts_ref, out_ref):
    acc = p_ref[0] * wts_ref[:, 0:1]
    for e in range(1, _E):
        acc += p_ref[e] * wts_ref[:, e:e + 1]
    out_ref[:, 0, :] = acc


def kernel(x, gate_w, expert_w):
    B_, S_, H = x.shape
    xf = x.reshape(B_ * S_, H)
    wts = _router(xf, gate_w.T, expert_w)
    partials = pl.pallas_call(
        _partials_body,
        grid=(_E, _H // _H_BS),
        in_specs=[
            pl.BlockSpec((_T, _H), lambda e, h: (0, 0)),
            pl.BlockSpec((1, _H_BS, _O), lambda e, h: (e, h, 0)),
        ],
        out_specs=pl.BlockSpec((1, _T, _O), lambda e, h: (e, 0, 0)),
        out_shape=jax.ShapeDtypeStruct((_E, _T, _O), jnp.float32),
        compiler_params=pltpu.CompilerParams(
            dimension_semantics=("arbitrary", "arbitrary")),
    )(xf, expert_w)
    out = pl.pallas_call(
        _combine_body,
        grid=(_O // _C_BS,),
        in_specs=[
            pl.BlockSpec((_E, _T, _C_BS), lambda o: (0, 0, o)),
            pl.BlockSpec((_T, _L), lambda o: (0, 0)),
        ],
        out_specs=pl.BlockSpec((_T, 1, _C_BS), lambda o: (0, 0, o)),
        out_shape=jax.ShapeDtypeStruct((_T, 1, _O), jnp.float32),
        compiler_params=pltpu.CompilerParams(
            dimension_semantics=("arbitrary",)),
    )(partials, wts)
    return out.reshape(B_, S_, _O)


# speedup vs baseline: 1.2892x; 1.1814x over previous
import jax
import jax.numpy as jnp
from jax.experimental import pallas as pl
from jax.experimental.pallas import tpu as pltpu

_E = 8
_T = 32
_H = 4096
_O = 4096
_H_BS = 512


def _moe_body(x_ref, gate_ref, w_ref, out_ref, wts_ref):
    e = pl.program_id(0)
    h = pl.program_id(1)

    @pl.when((e == 0) & (h == 0))
    def _compute_routing():
        logits = jnp.dot(x_ref[:, 0, :], gate_ref[...],
                         preferred_element_type=jnp.float32)
        idx = jax.lax.broadcasted_iota(jnp.int32, (_T, _E), 1)
        m1 = jnp.max(logits, axis=1, keepdims=True)
        i1 = jnp.min(jnp.where(logits == m1, idx, _E), axis=1, keepdims=True)
        masked = jnp.where(idx == i1, -jnp.inf, logits)
        m2 = jnp.max(masked, axis=1, keepdims=True)
        i2 = jnp.min(jnp.where(masked == m2, idx, _E), axis=1, keepdims=True)
        r = jnp.exp(m2 - m1)
        w1 = 1.0 / (1.0 + r)
        w2 = 1.0 - w1
        wts_ref[...] = (jnp.where(idx == i1, w1, 0.0)
                        + jnp.where(idx == i2, w2, 0.0))
        out_ref[...] = jnp.zeros_like(out_ref)

    xs = x_ref[:, 0, pl.ds(h * _H_BS, _H_BS)]
    contrib = jnp.dot(xs, w_ref[0], preferred_element_type=jnp.float32)
    idx = jax.lax.broadcasted_iota(jnp.int32, (_T, _E), 1)
    tw = jnp.sum(jnp.where(idx == e, wts_ref[...], 0.0),
                 axis=1, keepdims=True)
    out_ref[:, 0, :] += contrib * tw


def kernel(x, gate_w, expert_w):
    B_, S_, H = x.shape
    out = pl.pallas_call(
        _moe_body,
        grid=(_E, _H // _H_BS),
        in_specs=[
            pl.BlockSpec((_T, 1, _H), lambda e, h: (0, 0, 0)),
            pl.BlockSpec((_H, _E), lambda e, h: (0, 0)),
            pl.BlockSpec((1, _H_BS, _O), lambda e, h: (e, h, 0)),
        ],
        out_specs=pl.BlockSpec((_T, 1, _O), lambda e, h: (0, 0, 0)),
        out_shape=jax.ShapeDtypeStruct((_T, 1, _O), jnp.float32),
        scratch_shapes=[pltpu.VMEM((_T, _E), jnp.float32)],
        compiler_params=pltpu.CompilerParams(
            dimension_semantics=("arbitrary", "arbitrary")),
    )(x, gate_w, expert_w)
    return out


# speedup vs baseline: 1.2920x; 1.0022x over previous
import jax
import jax.numpy as jnp
from jax.experimental import pallas as pl
from jax.experimental.pallas import tpu as pltpu

_E = 8
_T = 32
_H = 4096
_O = 4096
_H_BS = 512


def _moe_body(x_ref, gate_ref, w_ref, out_ref, wts_ref):
    e = pl.program_id(0)
    h = pl.program_id(1)

    @pl.when((e == 0) & (h == 0))
    def _compute_routing():
        logits = jax.lax.dot_general(
            x_ref[:, 0, :], gate_ref[...], (((1,), (1,)), ((), ())),
            preferred_element_type=jnp.float32)
        idx = jax.lax.broadcasted_iota(jnp.int32, (_T, _E), 1)
        m1 = jnp.max(logits, axis=1, keepdims=True)
        i1 = jnp.min(jnp.where(logits == m1, idx, _E), axis=1, keepdims=True)
        masked = jnp.where(idx == i1, -jnp.inf, logits)
        m2 = jnp.max(masked, axis=1, keepdims=True)
        i2 = jnp.min(jnp.where(masked == m2, idx, _E), axis=1, keepdims=True)
        r = jnp.exp(m2 - m1)
        w1 = 1.0 / (1.0 + r)
        w2 = 1.0 - w1
        wts_ref[...] = (jnp.where(idx == i1, w1, 0.0)
                        + jnp.where(idx == i2, w2, 0.0))
        out_ref[...] = jnp.zeros_like(out_ref)

    xs = x_ref[:, 0, pl.ds(h * _H_BS, _H_BS)]
    contrib = jnp.dot(xs, w_ref[0], preferred_element_type=jnp.float32)
    idx = jax.lax.broadcasted_iota(jnp.int32, (_T, _E), 1)
    tw = jnp.sum(jnp.where(idx == e, wts_ref[...], 0.0),
                 axis=1, keepdims=True)
    out_ref[:, 0, :] += contrib * tw


def kernel(x, gate_w, expert_w):
    B_, S_, H = x.shape
    out = pl.pallas_call(
        _moe_body,
        grid=(_E, _H // _H_BS),
        in_specs=[
            pl.BlockSpec((_T, 1, _H), lambda e, h: (0, 0, 0)),
            pl.BlockSpec((_E, _H), lambda e, h: (0, 0)),
            pl.BlockSpec((1, _H_BS, _O), lambda e, h: (e, h, 0)),
        ],
        out_specs=pl.BlockSpec((_T, 1, _O), lambda e, h: (0, 0, 0)),
        out_shape=jax.ShapeDtypeStruct((_T, 1, _O), jnp.float32),
        scratch_shapes=[pltpu.VMEM((_T, _E), jnp.float32)],
        compiler_params=pltpu.CompilerParams(
            dimension_semantics=("arbitrary", "arbitrary")),
    )(x, gate_w.T, expert_w)
    return out
